# baseline (speedup 1.0000x reference)
import functools

import jax
import jax.numpy as jnp
from jax.experimental import pallas as pl
from jax.experimental.pallas import tpu as pltpu

B, S_TOK, D, H, E, K = 64, 4, 1024, 2048, 8, 2
NTOK = B * S_TOK
T = 128
NT = (NTOK * K) // T + (E - 1)
NSLOT = NT * T
H2 = H // 2

_INTERPRET = False


def _gelu(x):
    return 0.5 * x * (1.0 + jax.lax.erf(x * 0.7071067811865476))


def _router_body(x_ref, wg_ref, lb_ref, tok_ref, gate_ref, meta_ref):
    f32 = jnp.float32
    x = x_ref[...]
    wg = wg_ref[...]
    logits = jax.lax.dot_general(x, wg, (((1,), (1,)), ((), ())),
                                 preferred_element_type=f32)
    ei = jax.lax.broadcasted_iota(jnp.int32, (NTOK, E), 1)
    m1 = jnp.max(logits, axis=1, keepdims=True)
    i1 = jnp.min(jnp.where(logits == m1, ei, E), axis=1, keepdims=True)
    masked = jnp.where(ei == i1, jnp.float32(-1e30), logits)
    m2 = jnp.max(masked, axis=1, keepdims=True)
    i2 = jnp.min(jnp.where(masked == m2, ei, E), axis=1, keepdims=True)
    z = jnp.exp(m2 - m1)
    g1 = 1.0 / (1.0 + z)
    g2 = z / (1.0 + z)
    oh1 = (ei == i1).astype(f32)
    oh2 = (ei == i2).astype(f32)
    gates = oh1 * g1 + oh2 * g2
    usage = jnp.sum(gates, axis=0, keepdims=True) * (1.0 / NTOK)
    log_usage = jnp.maximum(jnp.log(usage), -1e9)
    lb_ref[...] = jnp.sum((1.0 / E) * (jnp.log(1.0 / E) - log_usage),
                          axis=1, keepdims=True)
    o12 = oh1 + oh2
    ti_r = jax.lax.broadcasted_iota(jnp.int32, (NTOK, NTOK), 0)
    ti_c = jax.lax.broadcasted_iota(jnp.int32, (NTOK, NTOK), 1)
    lower = (ti_c < ti_r).astype(f32)
    ex = jax.lax.dot_general(lower, o12, (((1,), (0,)), ((), ())),
                             preferred_element_type=f32)
    counts = jnp.sum(o12, axis=0, keepdims=True)
    ceils = jnp.floor((counts + (T - 1)) * (1.0 / T))
    ee_r = jax.lax.broadcasted_iota(jnp.int32, (E, E), 0)
    ee_c = jax.lax.broadcasted_iota(jnp.int32, (E, E), 1)
    mlt = (ee_r < ee_c).astype(f32)
    start = jax.lax.dot_general(ceils, mlt, (((1,), (0,)), ((), ())),
                                preferred_element_type=f32)
    base = start * T + ex
    pos1 = jnp.sum(oh1 * base, axis=1, keepdims=True)
    pos2 = jnp.sum(oh2 * base, axis=1, keepdims=True)
    si = jax.lax.broadcasted_iota(jnp.int32, (NTOK, NSLOT), 1)
    tvec = jax.lax.broadcasted_iota(jnp.int32, (NTOK, 1), 0).astype(f32)
    m1h = (si == pos1.astype(jnp.int32)).astype(f32)
    m2h = (si == pos2.astype(jnp.int32)).astype(f32)
    tok_sorted = jnp.sum(m1h * tvec + m2h * tvec, axis=0, keepdims=True)
    gate_sorted = jnp.sum(m1h * g1 + m2h * g2, axis=0, keepdims=True)
    tok_ref[...] = tok_sorted.astype(jnp.int32)
    gate_ref[...] = gate_sorted
    live = (counts > 0.0).astype(f32)
    n_live = jnp.sum(live)
    idx = jax.lax.dot_general(live, mlt, (((1,), (0,)), ((), ())),
                              preferred_element_type=f32)
    li8 = jax.lax.broadcasted_iota(jnp.int32, (1, E), 1).astype(f32)
    ordv = jnp.zeros((1, E), f32)
    stav = jnp.zeros((1, E), f32)
    ntlv = jnp.zeros((1, E), f32)
    for e in range(E):
        cond = (li8 == idx[0, e]) & (live[0, e] > 0.0)
        ordv = jnp.where(cond, float(e), ordv)
        stav = jnp.where(cond, start[0, e], stav)
        ntlv = jnp.where(cond, ceils[0, e], ntlv)
    meta = jnp.concatenate(
        [ordv, stav, ntlv, jnp.full((1, E), n_live, f32)], axis=1)
    meta_ref[...] = meta.astype(jnp.int32)


def _ffn_body(m_ref, x_ref, w1_hbm, w2_hbm, w3_hbm, b1_ref, b2_ref, b3_ref,
              tok_ref, gate_ref, out_ref, xb_s, w1_s, w2a_s, w2b_s, w3_s,
              sem):
    bf16 = jnp.bfloat16
    f32 = jnp.float32
    j = pl.program_id(0)
    n_live = m_ref[3 * E]
    e = m_ref[j]
    sta = m_ref[E + j]
    ntl = m_ref[2 * E + j]
    p = jax.lax.rem(j, 2)

    def cp_w1(src_e, slot):
        return pltpu.make_async_copy(w1_hbm.at[src_e], w1_s.at[slot],
                                     sem.at[slot])

    def cp_w2a(src_e):
        return pltpu.make_async_copy(w2_hbm.at[src_e, pl.ds(0, H2)], w2a_s,
                                     sem.at[2])

    def cp_w2b(src_e):
        return pltpu.make_async_copy(w2_hbm.at[src_e, pl.ds(H2, H2)], w2b_s,
                                     sem.at[3])

    def cp_w3(src_e, slot):
        return pltpu.make_async_copy(w3_hbm.at[src_e], w3_s.at[slot],
                                     sem.at[4 + slot])

    @pl.when(j == 0)
    def _prologue():
        out_ref[...] = jnp.zeros((NTOK, D), f32)
        xb_s[...] = x_ref[...].astype(bf16)
        cp_w1(e, 0).start()
        cp_w2a(e).start()
        cp_w2b(e).start()
        cp_w3(e, 0).start()

    @pl.when(j + 1 < n_live)
    def _prefetch():
        e2 = m_ref[j + 1]
        cp_w1(e2, 1 - p).start()
        cp_w3(e2, 1 - p).start()

    @pl.when(j < n_live)
    def _compute():
        cp_w1(e, p).wait()
        cp_w2a(e).wait()
        cp_w2b(e).wait()
        cp_w3(e, p).wait()
        b1s = jnp.reshape(b1_ref[pl.ds(e, 1), :], (1, H))
        b2s = jnp.reshape(b2_ref[pl.ds(e, 1), :], (1, H))
        b3s = jnp.reshape(b3_ref[pl.ds(e, 1), :], (1, D))
        def tile_body(k, carry):
            w1 = w1_s[p].astype(bf16)
            w2a = w2a_s[...].astype(bf16)
            w2b = w2b_s[...].astype(bf16)
            w3 = w3_s[p].astype(bf16)
            row = sta + k
            ids = tok_ref[row, :]
            pg = (jnp.reshape(ids, (T, 1))
                  == jax.lax.broadcasted_iota(jnp.int32, (T, NTOK), 1))
            xg = jnp.dot(pg.astype(bf16), xb_s[...],
                         preferred_element_type=f32)
            h1 = jax.lax.dot_general(xg.astype(bf16), w1,
                                     (((1,), (1,)), ((), ())),
                                     preferred_element_type=f32)
            h1 = _gelu(h1 + b1s).astype(bf16)
            h2a = jax.lax.dot_general(h1, w2a, (((1,), (1,)), ((), ())),
                                      preferred_element_type=f32)
            h2b = jax.lax.dot_general(h1, w2b, (((1,), (1,)), ((), ())),
                                      preferred_element_type=f32)
            za = _gelu(h2a + b2s[:, :H2]).astype(bf16)
            zb = _gelu(h2b + b2s[:, H2:]).astype(bf16)
            op = (jax.lax.dot_general(za, w3[:, :H2], (((1,), (1,)), ((), ())),
                                      preferred_element_type=f32)
                  + jax.lax.dot_general(zb, w3[:, H2:], (((1,), (1,)), ((), ())),
                                        preferred_element_type=f32))
            g = jnp.reshape(gate_ref[row, :], (T, 1))
            c = ((op + b3s) * g).astype(bf16)
            p2 = (jax.lax.broadcasted_iota(jnp.int32, (NTOK, T), 0)
                  == jnp.reshape(ids, (1, T)))
            out_ref[...] += jnp.dot(p2.astype(bf16), c,
                                    preferred_element_type=f32)
            return carry

        jax.lax.fori_loop(0, ntl, tile_body, 0)

        @pl.when(j + 1 < n_live)
        def _refill_w2():
            e2 = m_ref[j + 1]
            cp_w2a(e2).start()
            cp_w2b(e2).start()


@functools.partial(jax.jit, static_argnums=())
def kernel(input_tensor, Wg, W1, b1, W2, b2, W3, b3):
    x2d = input_tensor.reshape(NTOK, D)
    lb, tok, gate, meta = pl.pallas_call(
        _router_body,
        out_shape=[
            jax.ShapeDtypeStruct((1, 1), jnp.float32),
            jax.ShapeDtypeStruct((1, NSLOT), jnp.int32),
            jax.ShapeDtypeStruct((1, NSLOT), jnp.float32),
            jax.ShapeDtypeStruct((1, 4 * E), jnp.int32),
        ],
        interpret=_INTERPRET,
    )(x2d, Wg)
    tok2 = tok.reshape(NT, T)
    gate2 = gate.reshape(NT, T)
    meta_flat = meta.reshape(4 * E)

    grid_spec = pltpu.PrefetchScalarGridSpec(
        num_scalar_prefetch=1,
        grid=(E,),
        in_specs=[
            pl.BlockSpec((NTOK, D), lambda j, m: (0, 0)),
            pl.BlockSpec(memory_space=pltpu.MemorySpace.HBM),
            pl.BlockSpec(memory_space=pltpu.MemorySpace.HBM),
            pl.BlockSpec(memory_space=pltpu.MemorySpace.HBM),
            pl.BlockSpec((E, H), lambda j, m: (0, 0)),
            pl.BlockSpec((E, H), lambda j, m: (0, 0)),
            pl.BlockSpec((E, D), lambda j, m: (0, 0)),
            pl.BlockSpec((NT, T), lambda j, m: (0, 0)),
            pl.BlockSpec((NT, T), lambda j, m: (0, 0)),
        ],
        out_specs=pl.BlockSpec((NTOK, D), lambda j, m: (0, 0)),
        scratch_shapes=[
            pltpu.VMEM((NTOK, D), jnp.bfloat16),
            pltpu.VMEM((2, H, D), jnp.float32),
            pltpu.VMEM((H2, H), jnp.float32),
            pltpu.VMEM((H2, H), jnp.float32),
            pltpu.VMEM((2, D, H), jnp.float32),
            pltpu.SemaphoreType.DMA((6,)),
        ],
    )
    out2d = pl.pallas_call(
        _ffn_body,
        grid_spec=grid_spec,
        out_shape=jax.ShapeDtypeStruct((NTOK, D), jnp.float32),
        interpret=_INTERPRET,
    )(meta_flat, x2d, W1, W2, W3, b1, b2, b3, tok2, gate2)
    return out2d.reshape(B, S_TOK, D), lb[0, 0]


# speedup vs baseline: 1.0015x; 1.0015x over previous
import functools

import jax
import jax.numpy as jnp
from jax.experimental import pallas as pl
from jax.experimental.pallas import tpu as pltpu

B, S_TOK, D, H, E, K = 64, 4, 1024, 2048, 8, 2
NTOK = B * S_TOK
T = 128
NT = (NTOK * K) // T + (E - 1)
NSLOT = NT * T
H2 = H // 2

_INTERPRET = False


def _gelu(x):
    return 0.5 * x * (1.0 + jax.lax.erf(x * 0.7071067811865476))


def _router_body(x_ref, wg_ref, lb_ref, tok_ref, gate_ref, meta_ref):
    f32 = jnp.float32
    x = x_ref[...]
    wg = wg_ref[...]
    logits = jax.lax.dot_general(x, wg, (((1,), (1,)), ((), ())),
                                 preferred_element_type=f32)
    ei = jax.lax.broadcasted_iota(jnp.int32, (NTOK, E), 1)
    m1 = jnp.max(logits, axis=1, keepdims=True)
    i1 = jnp.min(jnp.where(logits == m1, ei, E), axis=1, keepdims=True)
    masked = jnp.where(ei == i1, jnp.float32(-1e30), logits)
    m2 = jnp.max(masked, axis=1, keepdims=True)
    i2 = jnp.min(jnp.where(masked == m2, ei, E), axis=1, keepdims=True)
    z = jnp.exp(m2 - m1)
    g1 = 1.0 / (1.0 + z)
    g2 = z / (1.0 + z)
    oh1 = (ei == i1).astype(f32)
    oh2 = (ei == i2).astype(f32)
    gates = oh1 * g1 + oh2 * g2
    usage = jnp.sum(gates, axis=0, keepdims=True) * (1.0 / NTOK)
    log_usage = jnp.maximum(jnp.log(usage), -1e9)
    lb_ref[...] = jnp.sum((1.0 / E) * (jnp.log(1.0 / E) - log_usage),
                          axis=1, keepdims=True)
    o12 = oh1 + oh2
    ti_r = jax.lax.broadcasted_iota(jnp.int32, (NTOK, NTOK), 0)
    ti_c = jax.lax.broadcasted_iota(jnp.int32, (NTOK, NTOK), 1)
    lower = (ti_c < ti_r).astype(f32)
    ex = jax.lax.dot_general(lower, o12, (((1,), (0,)), ((), ())),
                             preferred_element_type=f32)
    counts = jnp.sum(o12, axis=0, keepdims=True)
    ceils = jnp.floor((counts + (T - 1)) * (1.0 / T))
    ee_r = jax.lax.broadcasted_iota(jnp.int32, (E, E), 0)
    ee_c = jax.lax.broadcasted_iota(jnp.int32, (E, E), 1)
    mlt = (ee_r < ee_c).astype(f32)
    start = jax.lax.dot_general(ceils, mlt, (((1,), (0,)), ((), ())),
                                preferred_element_type=f32)
    base = start * T + ex
    pos1 = jnp.sum(oh1 * base, axis=1, keepdims=True)
    pos2 = jnp.sum(oh2 * base, axis=1, keepdims=True)
    si = jax.lax.broadcasted_iota(jnp.int32, (NTOK, NSLOT), 1)
    tvec = jax.lax.broadcasted_iota(jnp.int32, (NTOK, 1), 0).astype(f32)
    m1h = (si == pos1.astype(jnp.int32)).astype(f32)
    m2h = (si == pos2.astype(jnp.int32)).astype(f32)
    tok_sorted = jnp.sum(m1h * tvec + m2h * tvec, axis=0, keepdims=True)
    gate_sorted = jnp.sum(m1h * g1 + m2h * g2, axis=0, keepdims=True)
    tok_ref[...] = tok_sorted.astype(jnp.int32)
    gate_ref[...] = gate_sorted
    live = (counts > 0.0).astype(f32)
    n_live = jnp.sum(live)
    idx = jax.lax.dot_general(live, mlt, (((1,), (0,)), ((), ())),
                              preferred_element_type=f32)
    li8 = jax.lax.broadcasted_iota(jnp.int32, (1, E), 1).astype(f32)
    ordv = jnp.zeros((1, E), f32)
    stav = jnp.zeros((1, E), f32)
    ntlv = jnp.zeros((1, E), f32)
    for e in range(E):
        cond = (li8 == idx[0, e]) & (live[0, e] > 0.0)
        ordv = jnp.where(cond, float(e), ordv)
        stav = jnp.where(cond, start[0, e], stav)
        ntlv = jnp.where(cond, ceils[0, e], ntlv)
    meta = jnp.concatenate(
        [ordv, stav, ntlv, jnp.full((1, E), n_live, f32)], axis=1)
    meta_ref[...] = meta.astype(jnp.int32)


def _ffn_body(m_ref, x_ref, w1_hbm, w2_hbm, w3_hbm, b1_ref, b2_ref, b3_ref,
              tok_ref, gate_ref, out_ref, xb_s, w1_s, w2a_s, w2b_s, w3_s,
              sem):
    bf16 = jnp.bfloat16
    f32 = jnp.float32
    j = pl.program_id(0)
    n_live = m_ref[3 * E]
    e = m_ref[j]
    sta = m_ref[E + j]
    ntl = m_ref[2 * E + j]
    p = jax.lax.rem(j, 2)

    class _Pair:
        def __init__(self, a, b):
            self._a, self._b = a, b

        def start(self):
            self._a.start()
            self._b.start()

        def wait(self):
            self._a.wait()
            self._b.wait()

    def _split(src, dst, s0, n):
        h = n // 2
        return _Pair(
            pltpu.make_async_copy(src.at[pl.ds(0, h)], dst.at[pl.ds(0, h)],
                                  sem.at[s0]),
            pltpu.make_async_copy(src.at[pl.ds(h, h)], dst.at[pl.ds(h, h)],
                                  sem.at[s0 + 1]))

    def cp_w1(src_e, slot):
        return _split(w1_hbm.at[src_e], w1_s.at[slot], 2 * slot, H)

    def cp_w2a(src_e):
        return _split(w2_hbm.at[src_e, pl.ds(0, H2)], w2a_s, 4, H2)

    def cp_w2b(src_e):
        return _split(w2_hbm.at[src_e, pl.ds(H2, H2)], w2b_s, 6, H2)

    def cp_w3(src_e, slot):
        return _Pair(
            pltpu.make_async_copy(w3_hbm.at[src_e, pl.ds(0, H2 // 2)],
                                  w3_s.at[slot, pl.ds(0, H2 // 2)],
                                  sem.at[8 + 2 * slot]),
            pltpu.make_async_copy(w3_hbm.at[src_e, pl.ds(H2 // 2, H2 // 2)],
                                  w3_s.at[slot, pl.ds(H2 // 2, H2 // 2)],
                                  sem.at[9 + 2 * slot]))

    @pl.when(j == 0)
    def _prologue():
        out_ref[...] = jnp.zeros((NTOK, D), f32)
        xb_s[...] = x_ref[...].astype(bf16)
        cp_w1(e, 0).start()
        cp_w2a(e).start()
        cp_w2b(e).start()
        cp_w3(e, 0).start()

    @pl.when(j + 1 < n_live)
    def _prefetch():
        e2 = m_ref[j + 1]
        cp_w1(e2, 1 - p).start()
        cp_w3(e2, 1 - p).start()

    @pl.when(j < n_live)
    def _compute():
        cp_w1(e, p).wait()
        cp_w2a(e).wait()
        cp_w2b(e).wait()
        cp_w3(e, p).wait()
        b1s = jnp.reshape(b1_ref[pl.ds(e, 1), :], (1, H))
        b2s = jnp.reshape(b2_ref[pl.ds(e, 1), :], (1, H))
        b3s = jnp.reshape(b3_ref[pl.ds(e, 1), :], (1, D))
        def tile_body(k, carry):
            w1 = w1_s[p].astype(bf16)
            w2a = w2a_s[...].astype(bf16)
            w2b = w2b_s[...].astype(bf16)
            w3 = w3_s[p].astype(bf16)
            row = sta + k
            ids = tok_ref[row, :]
            pg = (jnp.reshape(ids, (T, 1))
                  == jax.lax.broadcasted_iota(jnp.int32, (T, NTOK), 1))
            xg = jnp.dot(pg.astype(bf16), xb_s[...],
                         preferred_element_type=f32)
            h1 = jax.lax.dot_general(xg.astype(bf16), w1,
                                     (((1,), (1,)), ((), ())),
                                     preferred_element_type=f32)
            h1 = _gelu(h1 + b1s).astype(bf16)
            h2a = jax.lax.dot_general(h1, w2a, (((1,), (1,)), ((), ())),
                                      preferred_element_type=f32)
            h2b = jax.lax.dot_general(h1, w2b, (((1,), (1,)), ((), ())),
                                      preferred_element_type=f32)
            za = _gelu(h2a + b2s[:, :H2]).astype(bf16)
            zb = _gelu(h2b + b2s[:, H2:]).astype(bf16)
            op = (jax.lax.dot_general(za, w3[:, :H2], (((1,), (1,)), ((), ())),
                                      preferred_element_type=f32)
                  + jax.lax.dot_general(zb, w3[:, H2:], (((1,), (1,)), ((), ())),
                                        preferred_element_type=f32))
            g = jnp.reshape(gate_ref[row, :], (T, 1))
            c = ((op + b3s) * g).astype(bf16)
            p2 = (jax.lax.broadcasted_iota(jnp.int32, (NTOK, T), 0)
                  == jnp.reshape(ids, (1, T)))
            out_ref[...] += jnp.dot(p2.astype(bf16), c,
                                    preferred_element_type=f32)
            return carry

        jax.lax.fori_loop(0, ntl, tile_body, 0)

        @pl.when(j + 1 < n_live)
        def _refill_w2():
            e2 = m_ref[j + 1]
            cp_w2a(e2).start()
            cp_w2b(e2).start()


@functools.partial(jax.jit, static_argnums=())
def kernel(input_tensor, Wg, W1, b1, W2, b2, W3, b3):
    x2d = input_tensor.reshape(NTOK, D)
    lb, tok, gate, meta = pl.pallas_call(
        _router_body,
        out_shape=[
            jax.ShapeDtypeStruct((1, 1), jnp.float32),
            jax.ShapeDtypeStruct((1, NSLOT), jnp.int32),
            jax.ShapeDtypeStruct((1, NSLOT), jnp.float32),
            jax.ShapeDtypeStruct((1, 4 * E), jnp.int32),
        ],
        interpret=_INTERPRET,
    )(x2d, Wg)
    tok2 = tok.reshape(NT, T)
    gate2 = gate.reshape(NT, T)
    meta_flat = meta.reshape(4 * E)

    grid_spec = pltpu.PrefetchScalarGridSpec(
        num_scalar_prefetch=1,
        grid=(E,),
        in_specs=[
            pl.BlockSpec((NTOK, D), lambda j, m: (0, 0)),
            pl.BlockSpec(memory_space=pltpu.MemorySpace.HBM),
            pl.BlockSpec(memory_space=pltpu.MemorySpace.HBM),
            pl.BlockSpec(memory_space=pltpu.MemorySpace.HBM),
            pl.BlockSpec((E, H), lambda j, m: (0, 0)),
            pl.BlockSpec((E, H), lambda j, m: (0, 0)),
            pl.BlockSpec((E, D), lambda j, m: (0, 0)),
            pl.BlockSpec((NT, T), lambda j, m: (0, 0)),
            pl.BlockSpec((NT, T), lambda j, m: (0, 0)),
        ],
        out_specs=pl.BlockSpec((NTOK, D), lambda j, m: (0, 0)),
        scratch_shapes=[
            pltpu.VMEM((NTOK, D), jnp.bfloat16),
            pltpu.VMEM((2, H, D), jnp.float32),
            pltpu.VMEM((H2, H), jnp.float32),
            pltpu.VMEM((H2, H), jnp.float32),
            pltpu.VMEM((2, D, H), jnp.float32),
            pltpu.SemaphoreType.DMA((12,)),
        ],
    )
    out2d = pl.pallas_call(
        _ffn_body,
        grid_spec=grid_spec,
        out_shape=jax.ShapeDtypeStruct((NTOK, D), jnp.float32),
        interpret=_INTERPRET,
    )(meta_flat, x2d, W1, W2, W3, b1, b2, b3, tok2, gate2)
    return out2d.reshape(B, S_TOK, D), lb[0, 0]


# speedup vs baseline: 1.1151x; 1.1135x over previous
import functools

import jax
import jax.numpy as jnp
from jax.experimental import pallas as pl
from jax.experimental.pallas import tpu as pltpu

B, S_TOK, D, H, E, K = 64, 4, 1024, 2048, 8, 2
NTOK = B * S_TOK
T = 128
NT = (NTOK * K) // T + (E - 1)
NSLOT = NT * T
HB = 2
HBLK = H // HB
NSTAGE = HB + 1


def _gelu(x):
    return 0.5 * x * (1.0 + jax.lax.erf(x * 0.7071067811865476))


def _router_body(x_ref, wg_ref, lb_ref, tok_ref, gate_ref, meta_ref):
    f32 = jnp.float32
    x = x_ref[...]
    wg = wg_ref[...]
    logits = jax.lax.dot_general(x, wg, (((1,), (1,)), ((), ())),
                                 preferred_element_type=f32)
    ei = jax.lax.broadcasted_iota(jnp.int32, (NTOK, E), 1)
    m1 = jnp.max(logits, axis=1, keepdims=True)
    i1 = jnp.min(jnp.where(logits == m1, ei, E), axis=1, keepdims=True)
    masked = jnp.where(ei == i1, jnp.float32(-1e30), logits)
    m2 = jnp.max(masked, axis=1, keepdims=True)
    i2 = jnp.min(jnp.where(masked == m2, ei, E), axis=1, keepdims=True)
    z = jnp.exp(m2 - m1)
    g1 = 1.0 / (1.0 + z)
    g2 = z / (1.0 + z)
    oh1 = (ei == i1).astype(f32)
    oh2 = (ei == i2).astype(f32)
    gates = oh1 * g1 + oh2 * g2
    usage = jnp.sum(gates, axis=0, keepdims=True) * (1.0 / NTOK)
    log_usage = jnp.maximum(jnp.log(usage), -1e9)
    lb_ref[...] = jnp.sum((1.0 / E) * (jnp.log(1.0 / E) - log_usage),
                          axis=1, keepdims=True)
    o12 = oh1 + oh2
    ti_r = jax.lax.broadcasted_iota(jnp.int32, (NTOK, NTOK), 0)
    ti_c = jax.lax.broadcasted_iota(jnp.int32, (NTOK, NTOK), 1)
    lower = (ti_c < ti_r).astype(f32)
    ex = jax.lax.dot_general(lower, o12, (((1,), (0,)), ((), ())),
                             preferred_element_type=f32)
    counts = jnp.sum(o12, axis=0, keepdims=True)
    ceils = jnp.floor((counts + (T - 1)) * (1.0 / T))
    ee_r = jax.lax.broadcasted_iota(jnp.int32, (E, E), 0)
    ee_c = jax.lax.broadcasted_iota(jnp.int32, (E, E), 1)
    mlt = (ee_r < ee_c).astype(f32)
    start = jax.lax.dot_general(ceils, mlt, (((1,), (0,)), ((), ())),
                                preferred_element_type=f32)
    n_tiles = jnp.sum(ceils)
    base = start * T + ex
    pos1 = jnp.sum(oh1 * base, axis=1, keepdims=True)
    pos2 = jnp.sum(oh2 * base, axis=1, keepdims=True)
    si = jax.lax.broadcasted_iota(jnp.int32, (NTOK, NSLOT), 1)
    tvec = jax.lax.broadcasted_iota(jnp.int32, (NTOK, 1), 0).astype(f32)
    m1h = (si == pos1.astype(jnp.int32)).astype(f32)
    m2h = (si == pos2.astype(jnp.int32)).astype(f32)
    tok_sorted = jnp.sum(m1h * tvec + m2h * tvec, axis=0, keepdims=True)
    gate_sorted = jnp.sum(m1h * g1 + m2h * g2, axis=0, keepdims=True)
    tok_ref[...] = tok_sorted.astype(jnp.int32)
    gate_ref[...] = gate_sorted
    li = jax.lax.broadcasted_iota(jnp.int32, (1, 16), 1).astype(f32)
    i_eff = jnp.minimum(li, n_tiles - 1.0)
    e_of = jnp.zeros((1, 16), f32)
    for e in range(E):
        s_e = start[0, e]
        c_e = ceils[0, e]
        e_of = jnp.where((i_eff >= s_e) & (i_eff < s_e + c_e), float(e), e_of)
    meta_vals = jnp.where(li < NT, e_of, n_tiles)
    meta_ref[...] = meta_vals.astype(jnp.int32)


def _ffn_body(m_ref, x_ref, w1_ref, w2_ref, w3_ref, b1_ref, b2_ref, b3_ref,
              tok_ref, gate_ref, out_ref, xg_s, h2_s):
    f32 = jnp.float32
    s = pl.program_id(0)
    t = pl.program_id(1)
    n_tiles = m_ref[NT]
    e = m_ref[t]
    row0 = t * T

    @pl.when(jnp.logical_and(s == 0, t == 0))
    def _init():
        out_ref[...] = jnp.zeros((NTOK, D), f32)

    valid = t < n_tiles

    @pl.when(jnp.logical_and(valid, s < HB))
    def _stage_a():
        @pl.when(s == 0)
        def _gather():
            ids = tok_ref[t, :]
            p = (jnp.reshape(ids, (T, 1))
                 == jax.lax.broadcasted_iota(jnp.int32, (T, NTOK), 1))
            xg = jnp.dot(p.astype(jnp.bfloat16), x_ref[...].astype(jnp.bfloat16),
                         preferred_element_type=f32)
            xg_s[pl.ds(row0, T), :] = xg.astype(jnp.bfloat16)

        h1 = jax.lax.dot_general(xg_s[pl.ds(row0, T), :],
                                 w1_ref[0].astype(jnp.bfloat16),
                                 (((1,), (1,)), ((), ())),
                                 preferred_element_type=f32)
        b1s = jnp.reshape(b1_ref[pl.ds(e, 1), pl.ds(s * HBLK, HBLK)], (1, HBLK))
        h1 = _gelu(h1 + b1s).astype(jnp.bfloat16)
        l2 = jax.lax.dot_general(h1, w2_ref[0].astype(jnp.bfloat16),
                                 (((1,), (1,)), ((), ())),
                                 preferred_element_type=f32)
        cur = h2_s[pl.ds(row0, T), :]
        h2_s[pl.ds(row0, T), :] = jnp.where(s == 0, l2, cur + l2)

    @pl.when(jnp.logical_and(valid, s == HB))
    def _stage_b():
        b2s = jnp.reshape(b2_ref[pl.ds(e, 1), :], (1, H))
        zz = h2_s[pl.ds(row0, T), :] + b2s
        a2 = _gelu(zz).astype(jnp.bfloat16)
        op = jax.lax.dot_general(a2, w3_ref[0].astype(jnp.bfloat16),
                                 (((1,), (1,)), ((), ())),
                                 preferred_element_type=f32)
        b3s = jnp.reshape(b3_ref[pl.ds(e, 1), :], (1, D))
        g = jnp.reshape(gate_ref[t, :], (T, 1))
        c = ((op + b3s) * g).astype(jnp.bfloat16)
        ids = tok_ref[t, :]
        p2 = (jax.lax.broadcasted_iota(jnp.int32, (NTOK, T), 0)
              == jnp.reshape(ids, (1, T)))
        out_ref[...] += jnp.dot(p2.astype(jnp.bfloat16), c,
                                preferred_element_type=f32)


@functools.partial(jax.jit, static_argnums=())
def kernel(input_tensor, Wg, W1, b1, W2, b2, W3, b3):
    x2d = input_tensor.reshape(NTOK, D)
    lb, tok, gate, meta = pl.pallas_call(
        _router_body,
        out_shape=[
            jax.ShapeDtypeStruct((1, 1), jnp.float32),
            jax.ShapeDtypeStruct((1, NSLOT), jnp.int32),
            jax.ShapeDtypeStruct((1, NSLOT), jnp.float32),
            jax.ShapeDtypeStruct((1, 16), jnp.int32),
        ],
    )(x2d, Wg)
    tok2 = tok.reshape(NT, T)
    gate2 = gate.reshape(NT, T)
    meta_flat = meta.reshape(16)

    grid_spec = pltpu.PrefetchScalarGridSpec(
        num_scalar_prefetch=1,
        grid=(NSTAGE, NT),
        in_specs=[
            pl.BlockSpec((NTOK, D), lambda s, t, m: (0, 0)),
            pl.BlockSpec((1, HBLK, D),
                         lambda s, t, m: (jnp.where(s < HB, m[t], m[NT - 1]),
                                          jnp.minimum(s, HB - 1), 0)),
            pl.BlockSpec((1, H, HBLK),
                         lambda s, t, m: (jnp.where(s < HB, m[t], m[NT - 1]),
                                          0, jnp.minimum(s, HB - 1))),
            pl.BlockSpec((1, D, H),
                         lambda s, t, m: (jnp.where(s >= HB, m[t], m[0]),
                                          0, 0)),
            pl.BlockSpec((E, H), lambda s, t, m: (0, 0)),
            pl.BlockSpec((E, H), lambda s, t, m: (0, 0)),
            pl.BlockSpec((E, D), lambda s, t, m: (0, 0)),
            pl.BlockSpec((NT, T), lambda s, t, m: (0, 0)),
            pl.BlockSpec((NT, T), lambda s, t, m: (0, 0)),
        ],
        out_specs=pl.BlockSpec((NTOK, D), lambda s, t, m: (0, 0)),
        scratch_shapes=[
            pltpu.VMEM((NSLOT, D), jnp.bfloat16),
            pltpu.VMEM((NSLOT, H), jnp.float32),
        ],
    )
    out2d = pl.pallas_call(
        _ffn_body,
        grid_spec=grid_spec,
        out_shape=jax.ShapeDtypeStruct((NTOK, D), jnp.float32),
    )(meta_flat, x2d, W1, W2, W3, b1, b2, b3, tok2, gate2)
    return out2d.reshape(B, S_TOK, D), lb[0, 0]
